# initial kernel scaffold (unmeasured)
import jax
import jax.numpy as jnp
from jax import lax
from jax.experimental import pallas as pl
from jax.experimental.pallas import tpu as pltpu


def kernel(
    x,
):
    def body(*refs):
        pass

    out_shape = jax.ShapeDtypeStruct(..., jnp.float32)
    return pl.pallas_call(body, out_shape=out_shape)(...)


# baseline (device time: 30982 ns/iter reference)
import jax
import jax.numpy as jnp
from jax import lax
from jax.experimental import pallas as pl
from jax.experimental.pallas import tpu as pltpu


def kernel(x):
    m, n = x.shape[2], x.shape[3]

    def body(x_ref, out_ref, comm_ref, send_sems, recv_sems):
        my_x = lax.axis_index("x")
        my_y = lax.axis_index("y")
        x_nbr = (1 - my_x, my_y)
        y_nbr = (my_x, 1 - my_y)

        barrier = pltpu.get_barrier_semaphore()
        for nbr in (x_nbr, y_nbr):
            pl.semaphore_signal(
                barrier, inc=1, device_id=nbr,
                device_id_type=pl.DeviceIdType.MESH,
            )
        pl.semaphore_wait(barrier, 2)

        rdma1 = pltpu.make_async_remote_copy(
            src_ref=x_ref.at[0, 0],
            dst_ref=comm_ref.at[0],
            send_sem=send_sems.at[0],
            recv_sem=recv_sems.at[0],
            device_id=x_nbr,
            device_id_type=pl.DeviceIdType.MESH,
        )
        rdma1.start()
        rdma1.wait()
        out_ref[...] = x_ref[0, 0] + comm_ref[0]

        rdma2 = pltpu.make_async_remote_copy(
            src_ref=out_ref,
            dst_ref=comm_ref.at[1],
            send_sem=send_sems.at[1],
            recv_sem=recv_sems.at[1],
            device_id=y_nbr,
            device_id_type=pl.DeviceIdType.MESH,
        )
        rdma2.start()
        rdma2.wait()
        out_ref[...] = out_ref[...] + comm_ref[1]

    return pl.pallas_call(
        body,
        out_shape=jax.ShapeDtypeStruct((m, n), jnp.float32),
        in_specs=[pl.BlockSpec(memory_space=pltpu.VMEM)],
        out_specs=pl.BlockSpec(memory_space=pltpu.VMEM),
        scratch_shapes=[
            pltpu.VMEM((2, m, n), jnp.float32),
            pltpu.SemaphoreType.DMA((2,)),
            pltpu.SemaphoreType.DMA((2,)),
        ],
        compiler_params=pltpu.CompilerParams(collective_id=0),
    )(x)


# device time: 19747 ns/iter; 1.5689x vs baseline; 1.5689x over previous
import jax
import jax.numpy as jnp
from jax import lax
from jax.experimental import pallas as pl
from jax.experimental.pallas import tpu as pltpu


def kernel(x):
    m, n = x.shape[2], x.shape[3]
    h = m // 2

    def body(x_ref, out_ref, comm_ref, send_sems, recv_sems):
        my_x = lax.axis_index("x")
        my_y = lax.axis_index("y")
        x_nbr = (1 - my_x, my_y)
        y_nbr = (my_x, 1 - my_y)

        barrier = pltpu.get_barrier_semaphore()
        for nbr in (x_nbr, y_nbr):
            pl.semaphore_signal(
                barrier, inc=1, device_id=nbr,
                device_id_type=pl.DeviceIdType.MESH,
            )
        pl.semaphore_wait(barrier, 2)

        def exchange(src_ref, slot, nbr):
            return pltpu.make_async_remote_copy(
                src_ref=src_ref,
                dst_ref=comm_ref.at[slot],
                send_sem=send_sems.at[slot],
                recv_sem=recv_sems.at[slot],
                device_id=nbr,
                device_id_type=pl.DeviceIdType.MESH,
            )

        a1 = exchange(x_ref.at[0, 0, 0:h], 0, x_nbr)
        b1 = exchange(x_ref.at[0, 0, h:m], 1, y_nbr)
        a1.start()
        b1.start()

        a1.wait_recv()
        out_ref[0:h, :] = x_ref[0, 0, 0:h, :] + comm_ref[0]
        a2 = exchange(out_ref.at[0:h], 2, y_nbr)
        a2.start()

        b1.wait_recv()
        out_ref[h:m, :] = x_ref[0, 0, h:m, :] + comm_ref[1]
        b2 = exchange(out_ref.at[h:m], 3, x_nbr)
        b2.start()

        a2.wait_recv()
        a2.wait_send()
        out_ref[0:h, :] = out_ref[0:h, :] + comm_ref[2]

        b2.wait_recv()
        b2.wait_send()
        out_ref[h:m, :] = out_ref[h:m, :] + comm_ref[3]

        a1.wait_send()
        b1.wait_send()

    return pl.pallas_call(
        body,
        out_shape=jax.ShapeDtypeStruct((m, n), jnp.float32),
        in_specs=[pl.BlockSpec(memory_space=pltpu.VMEM)],
        out_specs=pl.BlockSpec(memory_space=pltpu.VMEM),
        scratch_shapes=[
            pltpu.VMEM((4, h, n), jnp.float32),
            pltpu.SemaphoreType.DMA((4,)),
            pltpu.SemaphoreType.DMA((4,)),
        ],
        compiler_params=pltpu.CompilerParams(collective_id=0),
    )(x)


# device time: 18540 ns/iter; 1.6711x vs baseline; 1.0651x over previous
import jax
import jax.numpy as jnp
from jax import lax
from jax.experimental import pallas as pl
from jax.experimental.pallas import tpu as pltpu

N_CHUNKS = 4


def kernel(x):
    m, n = x.shape[2], x.shape[3]
    rows = m // N_CHUNKS

    def body(x_ref, out_ref, comm_ref, send_sems, recv_sems):
        my_x = lax.axis_index("x")
        my_y = lax.axis_index("y")
        x_nbr = (1 - my_x, my_y)
        y_nbr = (my_x, 1 - my_y)

        barrier = pltpu.get_barrier_semaphore()
        for nbr in (x_nbr, y_nbr):
            pl.semaphore_signal(
                barrier, inc=1, device_id=nbr,
                device_id_type=pl.DeviceIdType.MESH,
            )
        pl.semaphore_wait(barrier, 2)

        def rows_of(c):
            return slice(c * rows, (c + 1) * rows)

        def exchange(src_ref, slot, nbr):
            return pltpu.make_async_remote_copy(
                src_ref=src_ref,
                dst_ref=comm_ref.at[slot],
                send_sem=send_sems.at[slot],
                recv_sem=recv_sems.at[slot],
                device_id=nbr,
                device_id_type=pl.DeviceIdType.MESH,
            )

        def p1_nbr(c):
            return x_nbr if c < 2 else y_nbr

        def p2_nbr(c):
            return y_nbr if c < 2 else x_nbr

        p1 = [exchange(x_ref.at[0, 0, rows_of(c)], c, p1_nbr(c)) for c in range(4)]
        for c in (0, 2, 1, 3):
            p1[c].start()

        p2 = [None] * 4
        for c in (0, 2, 1, 3):
            p1[c].wait_recv()
            out_ref[rows_of(c), :] = x_ref[0, 0, rows_of(c), :] + comm_ref[c]
            p2[c] = exchange(out_ref.at[rows_of(c)], 4 + c, p2_nbr(c))
            p2[c].start()

        for c in (0, 2, 1, 3):
            p2[c].wait_recv()
            p2[c].wait_send()
            out_ref[rows_of(c), :] = out_ref[rows_of(c), :] + comm_ref[4 + c]

        for c in range(4):
            p1[c].wait_send()

    return pl.pallas_call(
        body,
        out_shape=jax.ShapeDtypeStruct((m, n), jnp.float32),
        in_specs=[pl.BlockSpec(memory_space=pltpu.VMEM)],
        out_specs=pl.BlockSpec(memory_space=pltpu.VMEM),
        scratch_shapes=[
            pltpu.VMEM((8, rows, n), jnp.float32),
            pltpu.SemaphoreType.DMA((8,)),
            pltpu.SemaphoreType.DMA((8,)),
        ],
        compiler_params=pltpu.CompilerParams(collective_id=0),
    )(x)


# device time: 15933 ns/iter; 1.9445x vs baseline; 1.1636x over previous
import jax
import jax.numpy as jnp
from jax import lax
from jax.experimental import pallas as pl
from jax.experimental.pallas import tpu as pltpu

Q = 128
K = 4
CH = Q // K


def kernel(x):
    m, n = x.shape[2], x.shape[3]
    half = m // 2

    def body(x_ref, out_ref, comm_ref, send_sems, recv_sems):
        my_x = lax.axis_index("x")
        my_y = lax.axis_index("y")
        x_nbr = (1 - my_x, my_y)
        y_nbr = (my_x, 1 - my_y)

        barrier = pltpu.get_barrier_semaphore()
        for nbr in (x_nbr, y_nbr):
            pl.semaphore_signal(
                barrier, inc=1, device_id=nbr,
                device_id_type=pl.DeviceIdType.MESH,
            )
        pl.semaphore_wait(barrier, 2)

        qa = my_x * Q
        sa = (1 - my_x) * Q
        qb = half + my_y * Q
        sb = half + (1 - my_y) * Q

        def swap(src_ref, dst_ref, slot, nbr):
            return pltpu.make_async_remote_copy(
                src_ref=src_ref,
                dst_ref=dst_ref,
                send_sem=send_sems.at[slot],
                recv_sem=recv_sems.at[slot],
                device_id=nbr,
                device_id_type=pl.DeviceIdType.MESH,
            )

        def ds(start, k):
            return pl.ds(start + k * CH, CH)

        s1a = [
            swap(x_ref.at[0, 0, ds(sa, k)], comm_ref.at[k], k, x_nbr)
            for k in range(K)
        ]
        s1b = [
            swap(x_ref.at[0, 0, ds(sb, k)], comm_ref.at[K + k], K + k, y_nbr)
            for k in range(K)
        ]
        for k in range(K):
            s1a[k].start()
            s1b[k].start()

        s2a, s2b = [None] * K, [None] * K
        for k in range(K):
            s1a[k].wait_recv()
            out_ref[ds(qa, k), :] = x_ref[0, 0, ds(qa, k), :] + comm_ref[k]
            s2a[k] = swap(
                out_ref.at[ds(qa, k)], comm_ref.at[2 * K + k], 2 * K + k, y_nbr
            )
            s2a[k].start()

            s1b[k].wait_recv()
            out_ref[ds(qb, k), :] = x_ref[0, 0, ds(qb, k), :] + comm_ref[K + k]
            s2b[k] = swap(
                out_ref.at[ds(qb, k)], comm_ref.at[3 * K + k], 3 * K + k, x_nbr
            )
            s2b[k].start()

        s3a, s3b = [None] * K, [None] * K
        for k in range(K):
            s2a[k].wait_recv()
            s2a[k].wait_send()
            out_ref[ds(qa, k), :] = out_ref[ds(qa, k), :] + comm_ref[2 * K + k]
            s3a[k] = swap(
                out_ref.at[ds(qa, k)], out_ref.at[ds(qa, k)], 4 * K + k, x_nbr
            )
            s3a[k].start()

            s2b[k].wait_recv()
            s2b[k].wait_send()
            out_ref[ds(qb, k), :] = out_ref[ds(qb, k), :] + comm_ref[3 * K + k]
            s3b[k] = swap(
                out_ref.at[ds(qb, k)], out_ref.at[ds(qb, k)], 5 * K + k, y_nbr
            )
            s3b[k].start()

        for k in range(K):
            s3a[k].wait_recv()
            s3b[k].wait_recv()
        for k in range(K):
            s1a[k].wait_send()
            s1b[k].wait_send()
            s3a[k].wait_send()
            s3b[k].wait_send()

    return pl.pallas_call(
        body,
        out_shape=jax.ShapeDtypeStruct((m, n), jnp.float32),
        in_specs=[pl.BlockSpec(memory_space=pltpu.VMEM)],
        out_specs=pl.BlockSpec(memory_space=pltpu.VMEM),
        scratch_shapes=[
            pltpu.VMEM((4 * K, CH, n), jnp.float32),
            pltpu.SemaphoreType.DMA((6 * K,)),
            pltpu.SemaphoreType.DMA((6 * K,)),
        ],
        compiler_params=pltpu.CompilerParams(collective_id=0),
    )(x)
